# baseline (speedup 1.0000x reference)
import functools

import jax
import jax.numpy as jnp
from jax import lax
from jax.experimental import pallas as pl
from jax.experimental.pallas import tpu as pltpu
from jax.experimental.pallas import tpu_sc as plsc

_BATCH = 16384
_NC = 1
_NS = plsc.get_sparse_core_info().num_subcores
_NW = _NC * _NS
_BPW = _BATCH // _NW
_CHUNK = _BPW // 2

_mesh = plsc.VectorSubcoreMesh(
    core_axis_name="c", subcore_axis_name="s", num_cores=_NC
)


@functools.partial(
    pl.kernel,
    mesh=_mesh,
    out_type=jax.ShapeDtypeStruct((_NW, 2, _CHUNK), jnp.float32),
    scratch_types=[
        pltpu.VMEM((_CHUNK,), jnp.int32),
        pltpu.VMEM((_CHUNK,), jnp.int32),
        pltpu.VMEM((_CHUNK,), jnp.float32),
        pltpu.VMEM((_CHUNK,), jnp.float32),
        pltpu.SemaphoreType.DMA,
        pltpu.SemaphoreType.DMA,
        pltpu.SemaphoreType.DMA,
        pltpu.SemaphoreType.DMA,
        pltpu.SemaphoreType.DMA,
        pltpu.SemaphoreType.DMA,
    ],
)
def _vtable_gather(
    idx_hbm, table_hbm, out_hbm,
    idx_v0, idx_v1, val_v0, val_v1,
    si0, si1, sg0, sg1, so0, so1,
):
    wid = lax.axis_index("s") * _NC + lax.axis_index("c")
    ci0 = pltpu.async_copy(idx_hbm.at[wid, 0], idx_v0, si0)
    ci1 = pltpu.async_copy(idx_hbm.at[wid, 1], idx_v1, si1)
    ci0.wait()
    g0 = pltpu.async_copy(table_hbm.at[idx_v0], val_v0, sg0)
    ci1.wait()
    g1 = pltpu.async_copy(table_hbm.at[idx_v1], val_v1, sg1)
    g0.wait()
    o0 = pltpu.async_copy(val_v0, out_hbm.at[wid, 0], so0)
    g1.wait()
    o1 = pltpu.async_copy(val_v1, out_hbm.at[wid, 1], so1)
    o0.wait()
    o1.wait()


def kernel(state, values):
    idx = state.astype(jnp.int32).reshape(_NW, 2, _CHUNK)
    out = _vtable_gather(idx, values)
    return out.reshape(_BATCH)[:, None]
